# Initial kernel scaffold; baseline (speedup 1.0000x reference)
import jax
import jax.numpy as jnp
from jax.experimental import pallas as pl


def kernel(hidden_states, weight, bias):
    raise NotImplementedError("write your pallas kernel here")


# speedup vs baseline: 4.3115x; 4.3115x over previous
import functools

import jax
import jax.numpy as jnp
from jax.experimental import pallas as pl

HIDDEN_DIM = 768
NUM_EXPERTS = 8
TOKENS = 32768
BLOCK_T = 2048


def _router_body(x_ref, wt_ref, b_ref, scores_ref, idx_ref):
    x = x_ref[...]
    wt = wt_ref[...]
    logits = jnp.dot(x, wt, preferred_element_type=jnp.float32)
    logits = logits + b_ref[...]

    iota = jax.lax.broadcasted_iota(jnp.int32, logits.shape, 1)
    m1 = jnp.max(logits, axis=1, keepdims=True)
    i1 = jnp.min(jnp.where(logits == m1, iota, NUM_EXPERTS), axis=1, keepdims=True)
    masked = jnp.where(iota == i1, -jnp.inf, logits)
    m2 = jnp.max(masked, axis=1, keepdims=True)
    i2 = jnp.min(jnp.where(masked == m2, iota, NUM_EXPERTS), axis=1, keepdims=True)

    d = jnp.exp(m2 - m1)
    p1 = 1.0 / (1.0 + d)
    p2 = 1.0 - p1

    scores_ref[...] = jnp.where(iota == i1, p1, jnp.where(iota == i2, p2, 0.0))
    idx_ref[...] = jnp.concatenate([i1, i2], axis=1)


@jax.jit
def kernel(hidden_states, weight, bias):
    x = hidden_states.reshape(-1, HIDDEN_DIM)
    t = x.shape[0]
    wt = weight.T
    b2 = bias.reshape(1, NUM_EXPERTS)
    grid = (t // BLOCK_T,)
    scores, indices = pl.pallas_call(
        _router_body,
        grid=grid,
        in_specs=[
            pl.BlockSpec((BLOCK_T, HIDDEN_DIM), lambda i: (i, 0)),
            pl.BlockSpec((HIDDEN_DIM, NUM_EXPERTS), lambda i: (0, 0)),
            pl.BlockSpec((1, NUM_EXPERTS), lambda i: (0, 0)),
        ],
        out_specs=[
            pl.BlockSpec((BLOCK_T, NUM_EXPERTS), lambda i: (i, 0)),
            pl.BlockSpec((BLOCK_T, 2), lambda i: (i, 0)),
        ],
        out_shape=[
            jax.ShapeDtypeStruct((t, NUM_EXPERTS), jnp.float32),
            jax.ShapeDtypeStruct((t, 2), jnp.int32),
        ],
    )(x, wt, b2)
    return scores, indices
